# Initial kernel scaffold; baseline (speedup 1.0000x reference)
import jax
import jax.numpy as jnp
from jax.experimental import pallas as pl


def kernel(e_pred, seg_mask):
    raise NotImplementedError("write your pallas kernel here")


# speedup vs baseline: 1.7393x; 1.7393x over previous
import functools

import jax
import jax.numpy as jnp
from jax import lax
from jax.experimental import pallas as pl
from jax.experimental.pallas import tpu as pltpu
from jax.experimental.pallas import tpu_sc as plsc

_NC = 2
_NS = 16
_L = 16
_NW = _NC * _NS
_N = 16 * 512 * 512
_PER_W = _N // _NW
_CHUNK = 8192
_ITERS = _CHUNK // _L
_NCHUNK = _PER_W // _CHUNK

_MU = (0.93, 0.9, 0.88, 0.85, 0.87, 0.85, 0.92, 0.91, 0.96, 0.95, 0.85,
       0.98, 0.97, 0.25, 0.3, 0.28, 0.27, 0.25, 0.28)
_SD = (0.03, 0.05, 0.06, 0.08, 0.07, 0.05, 0.04, 0.04, 0.02, 0.03, 0.1,
       0.01, 0.01, 0.1, 0.12, 0.11, 0.1, 0.1, 0.09)
_MARGIN = 1.5
_BIG = 1.0e9

_TBL = jnp.array(
    [m for m in _MU] + [0.0] * (32 - len(_MU))
    + [(_MARGIN * s if s < 0.12 else _BIG) for s in _SD]
    + [_BIG] * (32 - len(_SD)),
    dtype=jnp.float32)


def _sc_body(e_hbm, seg_hbm, tbl_hbm, out_hbm, mu_v, thr_v, e_v, s_v, o_v):
    cid = lax.axis_index("c")
    sid = lax.axis_index("s")
    wid = sid * _NC + cid
    base = wid * _PER_W

    pltpu.sync_copy(tbl_hbm.at[pl.ds(0, 32)], mu_v)
    pltpu.sync_copy(tbl_hbm.at[pl.ds(32, 32)], thr_v)

    def outer(ci, carry):
        acc_s, acc_c = carry
        off = base + ci * _CHUNK
        pltpu.sync_copy(e_hbm.at[pl.ds(off, _CHUNK)], e_v)
        pltpu.sync_copy(seg_hbm.at[pl.ds(off, _CHUNK)], s_v)

        def inner(i, c2):
            a_s, a_c = c2
            idx = s_v[pl.ds(i * _L, _L)]
            e = e_v[pl.ds(i * _L, _L)]
            mu = plsc.load_gather(mu_v, [idx])
            th = plsc.load_gather(thr_v, [idx])
            m = jnp.where(th < 1.0, 1.0, 0.0).astype(jnp.float32)
            p = jnp.maximum(jnp.abs(e - mu) - th, 0.0)
            return (a_s + p, a_c + m)

        return lax.fori_loop(0, _ITERS, inner, (acc_s, acc_c))

    zero = jnp.zeros((_L,), jnp.float32)
    acc_s, acc_c = lax.fori_loop(0, _NCHUNK, outer, (zero, zero))

    o_v[pl.ds(0, _L)] = acc_s
    pltpu.sync_copy(o_v, out_hbm.at[pl.ds(wid * _L, _L)])
    o_v[pl.ds(0, _L)] = acc_c
    pltpu.sync_copy(o_v, out_hbm.at[pl.ds(_NW * _L + wid * _L, _L)])


_sc_call = pl.kernel(
    _sc_body,
    out_type=jax.ShapeDtypeStruct((2 * _NW * _L,), jnp.float32),
    mesh=plsc.VectorSubcoreMesh(core_axis_name="c", subcore_axis_name="s"),
    compiler_params=pltpu.CompilerParams(needs_layout_passes=False),
    scratch_types=[
        pltpu.VMEM((32,), jnp.float32),
        pltpu.VMEM((32,), jnp.float32),
        pltpu.VMEM((_CHUNK,), jnp.float32),
        pltpu.VMEM((_CHUNK,), jnp.int32),
        pltpu.VMEM((_L,), jnp.float32),
    ],
)


def kernel(e_pred, seg_mask):
    e = e_pred.reshape(-1)
    seg = seg_mask.reshape(-1).astype(jnp.int32)
    part = _sc_call(e, seg, _TBL)
    psum = jnp.sum(part[: _NW * _L])
    total = jnp.sum(part[_NW * _L:])
    return jnp.where(total < 1.0, jnp.float32(0.0),
                     psum / jnp.maximum(total, 1.0))


# speedup vs baseline: 2.8691x; 1.6496x over previous
import jax
import jax.numpy as jnp
from jax import lax
from jax.experimental import pallas as pl
from jax.experimental.pallas import tpu as pltpu
from jax.experimental.pallas import tpu_sc as plsc

_NC = 2
_NS = 16
_L = 16
_NW = _NC * _NS
_N = 16 * 512 * 512
_PER_W = _N // _NW
_CHUNK = 16384
_NCHUNK = _PER_W // _CHUNK
_UNROLL = 8

_MU = (0.93, 0.9, 0.88, 0.85, 0.87, 0.85, 0.92, 0.91, 0.96, 0.95, 0.85,
       0.98, 0.97, 0.25, 0.3, 0.28, 0.27, 0.25, 0.28)
_SD = (0.03, 0.05, 0.06, 0.08, 0.07, 0.05, 0.04, 0.04, 0.02, 0.03, 0.1,
       0.01, 0.01, 0.1, 0.12, 0.11, 0.1, 0.1, 0.09)
_MARGIN = 1.5
_BIG = 1.0e9

_TBL = jnp.array(
    [(m - _MARGIN * s if s < 0.12 else -_BIG) for m, s in zip(_MU, _SD)]
    + [-_BIG] * (32 - len(_MU))
    + [(m + _MARGIN * s if s < 0.12 else _BIG) for m, s in zip(_MU, _SD)]
    + [_BIG] * (32 - len(_SD)),
    dtype=jnp.float32)


def _sc_body(e_hbm, seg_hbm, tbl_hbm, out_hbm,
             lo_v, hi_v, e_v0, e_v1, s_v0, s_v1, o_v,
             sem_e0, sem_e1, sem_s0, sem_s1):
    cid = lax.axis_index("c")
    sid = lax.axis_index("s")
    wid = sid * _NC + cid
    base = wid * _PER_W

    pltpu.sync_copy(tbl_hbm.at[pl.ds(0, 32)], lo_v)
    pltpu.sync_copy(tbl_hbm.at[pl.ds(32, 32)], hi_v)

    sem_e = (sem_e0, sem_e1)
    sem_s = (sem_s0, sem_s1)
    e_v = (e_v0, e_v1)
    s_v = (s_v0, s_v1)

    def fire(ci, b):
        off = base + ci * _CHUNK
        de = pltpu.async_copy(e_hbm.at[pl.ds(off, _CHUNK)], e_v[b],
                              sem_e[b])
        dg = pltpu.async_copy(seg_hbm.at[pl.ds(off, _CHUNK)], s_v[b],
                              sem_s[b])
        return de, dg

    def compute(b, acc):
        eb = e_v[b]
        sb = s_v[b]

        def body(i, c):
            out = []
            for j, (a_s, a_c) in enumerate(c):
                idx = sb[pl.ds(i + j * _L, _L)]
                e = eb[pl.ds(i + j * _L, _L)]
                lo = plsc.load_gather(lo_v, [idx])
                hi = plsc.load_gather(hi_v, [idx])
                p = jnp.maximum(jnp.maximum(lo - e, e - hi), 0.0)
                m = jnp.where(hi < 2.0, 1.0, 0.0).astype(jnp.float32)
                out.append((a_s + p, a_c + m))
            return tuple(out)

        return plsc.parallel_loop(0, _CHUNK, 2 * _L, unroll=_UNROLL,
                                  carry=acc)(body)

    zero = jnp.zeros((_L,), jnp.float32)
    acc = ((zero, zero), (zero, zero))
    prev = fire(0, 0)
    for ci in range(_NCHUNK):
        b = ci % 2
        nxt = fire(ci + 1, 1 - b) if ci + 1 < _NCHUNK else None
        prev[0].wait()
        prev[1].wait()
        acc = compute(b, acc)
        prev = nxt

    (s0, c0), (s1, c1) = acc
    o_v[pl.ds(0, _L)] = s0 + s1
    pltpu.sync_copy(o_v, out_hbm.at[pl.ds(wid * _L, _L)])
    o_v[pl.ds(0, _L)] = c0 + c1
    pltpu.sync_copy(o_v, out_hbm.at[pl.ds(_NW * _L + wid * _L, _L)])


_sc_call = pl.kernel(
    _sc_body,
    out_type=jax.ShapeDtypeStruct((2 * _NW * _L,), jnp.float32),
    mesh=plsc.VectorSubcoreMesh(core_axis_name="c", subcore_axis_name="s"),
    compiler_params=pltpu.CompilerParams(needs_layout_passes=False),
    scratch_types=[
        pltpu.VMEM((32,), jnp.float32),
        pltpu.VMEM((32,), jnp.float32),
        pltpu.VMEM((_CHUNK,), jnp.float32),
        pltpu.VMEM((_CHUNK,), jnp.float32),
        pltpu.VMEM((_CHUNK,), jnp.int32),
        pltpu.VMEM((_CHUNK,), jnp.int32),
        pltpu.VMEM((_L,), jnp.float32),
        pltpu.SemaphoreType.DMA,
        pltpu.SemaphoreType.DMA,
        pltpu.SemaphoreType.DMA,
        pltpu.SemaphoreType.DMA,
    ],
)


def kernel(e_pred, seg_mask):
    e = e_pred.reshape(-1)
    seg = seg_mask.reshape(-1).astype(jnp.int32)
    part = _sc_call(e, seg, _TBL)
    psum = jnp.sum(part[: _NW * _L])
    total = jnp.sum(part[_NW * _L:])
    return jnp.where(total < 1.0, jnp.float32(0.0),
                     psum / jnp.maximum(total, 1.0))
